# Initial kernel scaffold; baseline (speedup 1.0000x reference)
import jax
import jax.numpy as jnp
from jax.experimental import pallas as pl


def kernel(x, W, gamma, beta):
    raise NotImplementedError("write your pallas kernel here")


# speedup vs baseline: 15.1330x; 15.1330x over previous
import functools

import jax
import jax.numpy as jnp
from jax import lax
from jax.experimental import pallas as pl
from jax.experimental.pallas import tpu as pltpu
from jax.experimental.pallas import tpu_sc as plsc

_B, _C, _N, _K, _OUT = 8, 3, 4096, 16, 64
_R = 512
_NEGINF = float("-inf")

_NC, _NS = 2, 16
_NW = _NC * _NS
_PTS_W = (_B * _N) // _NW
_G = 8
_IDXG = _G * _K


def _topk_proj_body(xt_ref, x_ref, wg_ref, wq_ref, idx_ref, t_ref, q_ref):
    b = pl.program_id(0)
    i = pl.program_id(1)
    xt = xt_ref[0]
    x = x_ref[0]

    xth = xt.astype(jnp.bfloat16)
    xh = x.astype(jnp.bfloat16)
    inner = jnp.dot(xth, xh, preferred_element_type=jnp.float32)
    sqr = jnp.sum(xt * xt, axis=1, keepdims=True)
    sqc = jnp.sum(x * x, axis=0, keepdims=True)
    nd = 2.0 * inner - sqr - sqc

    rows = i * _R + lax.broadcasted_iota(jnp.int32, (_R, _N), 0)
    cols = lax.broadcasted_iota(jnp.int32, (_R, _N), 1)
    nd = jnp.where(rows == cols, _NEGINF, nd)

    parts = []
    for _ in range(_K):
        m = jnp.max(nd, axis=1, keepdims=True)
        am = jnp.min(jnp.where(nd == m, cols, _N), axis=1, keepdims=True)
        parts.append(am)
        nd = jnp.where(cols == am, _NEGINF, nd)
    idx_ref[0] = jnp.concatenate(parts, axis=1) + b * _N

    t = jnp.zeros((_R, _OUT), jnp.float32)
    q = jnp.zeros((_R, _OUT), jnp.float32)
    for c in range(_C):
        xc = xt[:, c:c + 1]
        t = t + xc * wg_ref[c:c + 1, :]
        q = q + xc * wq_ref[c:c + 1, :]
    t_ref[0] = jnp.concatenate([t, t * t], axis=1)
    q_ref[0] = q


_topk_proj = pl.pallas_call(
    _topk_proj_body,
    grid=(_B, _N // _R),
    in_specs=[
        pl.BlockSpec((1, _R, _C), lambda b, i: (b, i, 0)),
        pl.BlockSpec((1, _C, _N), lambda b, i: (b, 0, 0)),
        pl.BlockSpec((_C, _OUT), lambda b, i: (0, 0)),
        pl.BlockSpec((_C, _OUT), lambda b, i: (0, 0)),
    ],
    out_specs=[
        pl.BlockSpec((1, _R, _K), lambda b, i: (b, i, 0)),
        pl.BlockSpec((1, _R, 2 * _OUT), lambda b, i: (b, i, 0)),
        pl.BlockSpec((1, _R, _OUT), lambda b, i: (b, i, 0)),
    ],
    out_shape=[
        jax.ShapeDtypeStruct((_B, _N, _K), jnp.int32),
        jax.ShapeDtypeStruct((_B, _N, 2 * _OUT), jnp.float32),
        jax.ShapeDtypeStruct((_B, _N, _OUT), jnp.float32),
    ],
)


def _gather_reduce_body(t_hbm, idx_hbm, max_o, sum_o, sq_o,
                        idx_v, rows_v, mx_v, sm_v, sq_v, sem):
    wid = lax.axis_index("s") * _NC + lax.axis_index("c")
    base_pt = wid * _PTS_W

    def group_body(g, carry):
        pbase = base_pt + g * _G
        pltpu.sync_copy(idx_hbm.at[pl.ds(pbase * _K, _IDXG)], idx_v)
        pltpu.async_copy(t_hbm.at[idx_v], rows_v, sem).wait()

        def point_body(p, carry2):
            row0 = p * _K
            for c4 in range(_OUT // 16):
                sl = pl.ds(c4 * 16, 16)
                sl2 = pl.ds(_OUT + c4 * 16, 16)
                v = rows_v[row0, sl]
                mx, sm, sq = v, v, rows_v[row0, sl2]
                for r in range(1, _K):
                    v = rows_v[row0 + r, sl]
                    mx = jnp.maximum(mx, v)
                    sm = sm + v
                    sq = sq + rows_v[row0 + r, sl2]
                mx_v[p, sl] = mx
                sm_v[p, sl] = sm
                sq_v[p, sl] = sq
            return carry2

        lax.fori_loop(0, _G, point_body, 0)
        pltpu.sync_copy(mx_v, max_o.at[pl.ds(pbase, _G)])
        pltpu.sync_copy(sm_v, sum_o.at[pl.ds(pbase, _G)])
        pltpu.sync_copy(sq_v, sq_o.at[pl.ds(pbase, _G)])
        return carry

    lax.fori_loop(0, _PTS_W // _G, group_body, 0)


@functools.cache
def _make_gather_reduce():
    return functools.partial(
        pl.kernel,
        mesh=plsc.VectorSubcoreMesh(
            core_axis_name="c", subcore_axis_name="s",
            num_cores=_NC, num_subcores=_NS),
        out_type=[
            jax.ShapeDtypeStruct((_B * _N, _OUT), jnp.float32),
            jax.ShapeDtypeStruct((_B * _N, _OUT), jnp.float32),
            jax.ShapeDtypeStruct((_B * _N, _OUT), jnp.float32),
        ],
        scratch_types=[
            pltpu.VMEM((_IDXG,), jnp.int32),
            pltpu.VMEM((_IDXG, 2 * _OUT), jnp.float32),
            pltpu.VMEM((_G, _OUT), jnp.float32),
            pltpu.VMEM((_G, _OUT), jnp.float32),
            pltpu.VMEM((_G, _OUT), jnp.float32),
            pltpu.SemaphoreType.DMA,
        ],
    )(_gather_reduce_body)


_SCH = 4096


def _stats_body(sm_ref, sq_ref, q_ref, out_ref):
    i = pl.program_id(0)
    sm = sm_ref[...]
    sq = sq_ref[...]
    q = q_ref[...]
    blk = jnp.concatenate([
        jnp.sum(sm, axis=0, keepdims=True),
        jnp.sum(sq, axis=0, keepdims=True),
        jnp.sum(sm * q, axis=0, keepdims=True),
        jnp.sum(q, axis=0, keepdims=True),
        jnp.sum(q * q, axis=0, keepdims=True),
        jnp.zeros((3, _OUT), jnp.float32),
    ], axis=0)

    @pl.when(i == 0)
    def _():
        out_ref[...] = blk

    @pl.when(i > 0)
    def _():
        out_ref[...] = out_ref[...] + blk


_stats = pl.pallas_call(
    _stats_body,
    grid=((_B * _N) // _SCH,),
    in_specs=[
        pl.BlockSpec((_SCH, _OUT), lambda i: (i, 0)),
        pl.BlockSpec((_SCH, _OUT), lambda i: (i, 0)),
        pl.BlockSpec((_SCH, _OUT), lambda i: (i, 0)),
    ],
    out_specs=pl.BlockSpec((8, _OUT), lambda i: (0, 0)),
    out_shape=jax.ShapeDtypeStruct((8, _OUT), jnp.float32),
)


def _final_body(mx_ref, q_ref, a_ref, c_ref, o_ref):
    y = (mx_ref[...] + q_ref[...]) * a_ref[...] + c_ref[...]
    o_ref[...] = jnp.where(y > 0, y, 0.2 * y)


_final = pl.pallas_call(
    _final_body,
    grid=((_B * _N) // _SCH,),
    in_specs=[
        pl.BlockSpec((_SCH, _OUT), lambda i: (i, 0)),
        pl.BlockSpec((_SCH, _OUT), lambda i: (i, 0)),
        pl.BlockSpec((1, _OUT), lambda i: (0, 0)),
        pl.BlockSpec((1, _OUT), lambda i: (0, 0)),
    ],
    out_specs=pl.BlockSpec((_SCH, _OUT), lambda i: (i, 0)),
    out_shape=jax.ShapeDtypeStruct((_B * _N, _OUT), jnp.float32),
)


def kernel(x, W, gamma, beta):
    xt = jnp.transpose(x, (0, 2, 1))
    w1 = W[:, :_C]
    w2 = W[:, _C:]
    wg = jnp.transpose(w1)
    wq = jnp.transpose(w2 - w1)

    idx, t, q = _topk_proj(xt, x, wg, wq)
    t_flat = t.reshape(_B * _N, 2 * _OUT)
    q_flat = q.reshape(_B * _N, _OUT)
    idx_flat = idx.reshape(_B * _N * _K)

    mx, sm, sq = _make_gather_reduce()(t_flat, idx_flat)

    s = _stats(sm, sq, q_flat)
    m_tot = float(_B * _N * _K)
    mean = (s[0] + _K * s[3]) / m_tot
    e2 = (s[1] + 2.0 * s[2] + _K * s[4]) / m_tot
    var = e2 - mean * mean
    a = gamma / jnp.sqrt(var + 1e-5)
    c = beta - mean * a

    out_nm = _final(mx, q_flat, a[None, :], c[None, :])
    return out_nm.reshape(_B, _N, _OUT).transpose(0, 2, 1)
